# Initial kernel scaffold; baseline (speedup 1.0000x reference)
import jax
import jax.numpy as jnp
from jax.experimental import pallas as pl


def kernel(x, edge_index, W, b, u0):
    raise NotImplementedError("write your pallas kernel here")


# speedup vs baseline: 26.7991x; 26.7991x over previous
import functools

import jax
import jax.numpy as jnp
from jax import lax
from jax.experimental import pallas as pl
from jax.experimental.pallas import tpu as pltpu
from jax.experimental.pallas import tpu_sc as plsc

N_NODES = 10000
N_EDGES = 320000
D_FEAT = 128
D_HID = 128

NC = 2
NS = 16
NW = NC * NS
EPT = N_EDGES // NW
CH = 80
NCHUNK = EPT // CH
N_PAD = 10240
ROWS_PT = N_PAD // NS
DEGW = 16

_mesh = plsc.VectorSubcoreMesh(core_axis_name="c", subcore_axis_name="s")


@functools.partial(
    pl.kernel,
    out_type=jax.ShapeDtypeStruct((NC, N_PAD, DEGW), jnp.float32),
    mesh=_mesh,
    scratch_types=[
        pltpu.VMEM((NCHUNK, CH), jnp.int32),
        pltpu.VMEM((CH, DEGW), jnp.float32),
        pltpu.VMEM((16, DEGW), jnp.float32),
        pltpu.VMEM_SHARED((N_PAD, DEGW), jnp.float32),
    ],
)
def _deg_kernel(dst_hbm, out_hbm, idx_v, ones_v, zeros_v, acc):
    cid = lax.axis_index("c")
    sid = lax.axis_index("s")
    wid = cid * NS + sid

    one16 = jnp.ones((16,), jnp.float32)
    zero16 = jnp.zeros((16,), jnp.float32)
    for i in range(CH):
        ones_v[i, :] = one16
    for i in range(16):
        zeros_v[i, :] = zero16
    base = sid * ROWS_PT

    def zbody(j, carry):
        pltpu.sync_copy(zeros_v, acc.at[pl.ds(base + j * 16, 16)])
        return carry

    lax.fori_loop(0, ROWS_PT // 16, zbody, 0)
    pltpu.sync_copy(dst_hbm.at[wid], idx_v)
    plsc.subcore_barrier()

    def body(j, carry):
        pltpu.sync_copy(ones_v, acc.at[idx_v.at[j]], add=True)
        return carry

    lax.fori_loop(0, NCHUNK, body, 0)
    plsc.subcore_barrier()
    pltpu.sync_copy(acc.at[pl.ds(base, ROWS_PT)],
                    out_hbm.at[cid, pl.ds(base, ROWS_PT)])


@functools.partial(
    pl.kernel,
    out_type=jax.ShapeDtypeStruct((NC, N_PAD, D_HID), jnp.float32),
    mesh=_mesh,
    scratch_types=[
        pltpu.VMEM((NCHUNK, CH), jnp.int32),
        pltpu.VMEM((NCHUNK, CH), jnp.int32),
        pltpu.VMEM((CH, D_HID), jnp.float32),
        pltpu.VMEM((16, D_HID), jnp.float32),
        pltpu.VMEM_SHARED((N_PAD, D_HID), jnp.float32),
        pltpu.SemaphoreType.DMA,
    ],
)
def _edge_kernel(g_hbm, src_hbm, dst_hbm, out_hbm,
                 src_v, dst_v, rows_v, zeros_v, acc, sem):
    cid = lax.axis_index("c")
    sid = lax.axis_index("s")
    wid = cid * NS + sid

    zero16 = jnp.zeros((16,), jnp.float32)
    for i in range(16):
        for k in range(D_HID // 16):
            zeros_v[i, pl.ds(16 * k, 16)] = zero16
    base = sid * ROWS_PT

    def zbody(j, carry):
        pltpu.sync_copy(zeros_v, acc.at[pl.ds(base + j * 16, 16)])
        return carry

    lax.fori_loop(0, ROWS_PT // 16, zbody, 0)
    pltpu.sync_copy(src_hbm.at[wid], src_v)
    pltpu.sync_copy(dst_hbm.at[wid], dst_v)
    plsc.subcore_barrier()

    def body(j, carry):
        pltpu.async_copy(g_hbm.at[src_v.at[j]], rows_v, sem).wait()
        pltpu.sync_copy(rows_v, acc.at[dst_v.at[j]], add=True)
        return carry

    lax.fori_loop(0, NCHUNK, body, 0)
    plsc.subcore_barrier()
    pltpu.sync_copy(acc.at[pl.ds(base, ROWS_PT)],
                    out_hbm.at[cid, pl.ds(base, ROWS_PT)])


def _mm_body(x_ref, w_ref, u_ref, degp_ref, g_ref):
    W = w_ref[...]
    u = u_ref[...]
    v = None
    for _ in range(3):
        v = jnp.dot(u, W, preferred_element_type=jnp.float32)
        v = v / (jnp.sqrt(jnp.sum(v * v)) + 1e-12)
        u = lax.dot_general(v, W, (((1,), (1,)), ((), ())),
                            preferred_element_type=jnp.float32)
        u = u / (jnp.sqrt(jnp.sum(u * u)) + 1e-12)
    sigma = jnp.sum(jnp.dot(u, W, preferred_element_type=jnp.float32) * v)
    w_sn = W / sigma
    deg = (degp_ref[0, 0:N_NODES, 0:1]
           + degp_ref[1, 0:N_NODES, 0:1] + 1.0)
    d = lax.rsqrt(deg)
    g_ref[...] = jnp.dot(x_ref[...], w_sn,
                         preferred_element_type=jnp.float32) * d


def _combine_body(s_ref, g_ref, degp_ref, b_ref, out_ref):
    deg = degp_ref[0, :, 0:1] + degp_ref[1, :, 0:1] + 1.0
    d = lax.rsqrt(deg)
    out_ref[...] = d * (s_ref[0] + s_ref[1] + g_ref[...]) + b_ref[...]


_B4 = 1000


def kernel(x, edge_index, W, b, u0):
    src = edge_index[0].astype(jnp.int32).reshape(NW, NCHUNK, CH)
    dst = edge_index[1].astype(jnp.int32).reshape(NW, NCHUNK, CH)
    u0_2d = u0.reshape(1, D_FEAT).astype(jnp.float32)
    b_2d = b.reshape(1, D_HID).astype(jnp.float32)

    degp = _deg_kernel(dst)

    g = pl.pallas_call(
        _mm_body,
        out_shape=jax.ShapeDtypeStruct((N_NODES, D_HID), jnp.float32),
    )(x, W, u0_2d, degp)

    s = _edge_kernel(g, src, dst)

    out = pl.pallas_call(
        _combine_body,
        grid=(N_NODES // _B4,),
        in_specs=[
            pl.BlockSpec((NC, _B4, D_HID), lambda i: (0, i, 0)),
            pl.BlockSpec((_B4, D_HID), lambda i: (i, 0)),
            pl.BlockSpec((NC, _B4, DEGW), lambda i: (0, i, 0)),
            pl.BlockSpec((1, D_HID), lambda i: (0, 0)),
        ],
        out_specs=pl.BlockSpec((_B4, D_HID), lambda i: (i, 0)),
        out_shape=jax.ShapeDtypeStruct((N_NODES, D_HID), jnp.float32),
    )(s, g, degp, b_2d)
    return out


# speedup vs baseline: 33.5863x; 1.2533x over previous
import functools

import jax
import jax.numpy as jnp
from jax import lax
from jax.experimental import pallas as pl
from jax.experimental.pallas import tpu as pltpu
from jax.experimental.pallas import tpu_sc as plsc

N_NODES = 10000
N_EDGES = 320000
D_FEAT = 128
D_HID = 128

NC = 2
NS = 16
NW = NC * NS
EPT = N_EDGES // NW
CH = 128
NCHUNK = 79
EPT_P = NCHUNK * CH
PADE = EPT_P - EPT
N_PAD = 10112
ROWS_PT = N_PAD // NS
DEGW = 16

_mesh = plsc.VectorSubcoreMesh(core_axis_name="c", subcore_axis_name="s")


def _unpack(packed_v, j, srcc, dstc, b):
    for k in range(CH // 16):
        w = packed_v[j, pl.ds(16 * k, 16)]
        dstc[b, pl.ds(16 * k, 16)] = jnp.right_shift(w, 14)
        if srcc is not None:
            srcc[b, pl.ds(16 * k, 16)] = jnp.bitwise_and(w, 16383)


@functools.partial(
    pl.kernel,
    out_type=jax.ShapeDtypeStruct((NC, N_PAD, DEGW), jnp.float32),
    mesh=_mesh,
    scratch_types=[
        pltpu.VMEM((NCHUNK, CH), jnp.int32),
        pltpu.VMEM((CH, DEGW), jnp.float32),
        pltpu.VMEM((32, DEGW), jnp.float32),
        pltpu.VMEM((2, CH), jnp.int32),
        pltpu.VMEM_SHARED((N_PAD, DEGW), jnp.float32),
        [pltpu.SemaphoreType.DMA] * 2,
    ],
)
def _deg_kernel(pk_hbm, out_hbm, packed_v, ones_v, zeros_v, dstc, acc, ssems):
    cid = lax.axis_index("c")
    sid = lax.axis_index("s")
    wid = cid * NS + sid

    one16 = jnp.ones((16,), jnp.float32)
    zero16 = jnp.zeros((16,), jnp.float32)
    for i in range(CH):
        ones_v[i, :] = one16
    for i in range(32):
        zeros_v[i, :] = zero16
    base = sid * ROWS_PT

    def zbody(j, carry):
        pltpu.sync_copy(zeros_v, acc.at[pl.ds(base + j * 32, 32)])
        return carry

    lax.fori_loop(0, 19, zbody, 0)
    pltpu.sync_copy(zeros_v.at[pl.ds(0, 24)],
                    acc.at[pl.ds(base + 608, 24)])
    pltpu.sync_copy(pk_hbm.at[wid], packed_v)
    plsc.subcore_barrier()

    def body(j, carry):
        _unpack(packed_v, j, None, dstc, 0)
        pltpu.sync_copy(ones_v, acc.at[dstc.at[0]], add=True)
        return carry

    lax.fori_loop(0, NCHUNK, body, 0)
    plsc.subcore_barrier()
    pltpu.sync_copy(acc.at[pl.ds(base, ROWS_PT)],
                    out_hbm.at[cid, pl.ds(base, ROWS_PT)])


@functools.partial(
    pl.kernel,
    out_type=jax.ShapeDtypeStruct((NC, N_PAD, D_HID), jnp.float32),
    mesh=_mesh,
    scratch_types=[
        pltpu.VMEM((NCHUNK, CH), jnp.int32),
        pltpu.VMEM((2, CH), jnp.int32),
        pltpu.VMEM((2, CH), jnp.int32),
        pltpu.VMEM((2, CH, D_HID), jnp.float32),
        pltpu.VMEM((32, D_HID), jnp.float32),
        pltpu.VMEM_SHARED((N_PAD, D_HID), jnp.float32),
        [pltpu.SemaphoreType.DMA] * 2,
    ],
)
def _edge_kernel(g_hbm, pk_hbm, out_hbm,
                 packed_v, srcc, dstc, rows_v, zeros_v, acc, gsems):
    cid = lax.axis_index("c")
    sid = lax.axis_index("s")
    wid = cid * NS + sid

    zero16 = jnp.zeros((16,), jnp.float32)
    for i in range(32):
        for k in range(D_HID // 16):
            zeros_v[i, pl.ds(16 * k, 16)] = zero16
    base = sid * ROWS_PT

    def zbody(j, carry):
        pltpu.sync_copy(zeros_v, acc.at[pl.ds(base + j * 32, 32)])
        return carry

    lax.fori_loop(0, 19, zbody, 0)
    pltpu.sync_copy(zeros_v.at[pl.ds(0, 24)],
                    acc.at[pl.ds(base + 608, 24)])
    pltpu.sync_copy(pk_hbm.at[wid], packed_v)
    plsc.subcore_barrier()

    def gather(j, b):
        _unpack(packed_v, j, srcc, dstc, b)
        return pltpu.async_copy(g_hbm.at[srcc.at[b]], rows_v.at[b], gsems[b])

    def scatter(b):
        pltpu.sync_copy(rows_v.at[b], acc.at[dstc.at[b]], add=True)

    def body(t, carry):
        j = 2 * t
        d0 = gather(j, 0)
        d1 = gather(j + 1, 1)
        d0.wait()
        scatter(0)
        d1.wait()
        scatter(1)
        return carry

    lax.fori_loop(0, (NCHUNK - 1) // 2, body, 0)
    d = gather(NCHUNK - 1, 0)
    d.wait()
    scatter(0)
    plsc.subcore_barrier()
    pltpu.sync_copy(acc.at[pl.ds(base, ROWS_PT)],
                    out_hbm.at[cid, pl.ds(base, ROWS_PT)])


def _mm_body(x_ref, w_ref, u_ref, degp_ref, g_ref):
    W = w_ref[...]
    u = u_ref[...]
    v = None
    for _ in range(3):
        v = jnp.dot(u, W, preferred_element_type=jnp.float32)
        v = v / (jnp.sqrt(jnp.sum(v * v)) + 1e-12)
        u = lax.dot_general(v, W, (((1,), (1,)), ((), ())),
                            preferred_element_type=jnp.float32)
        u = u / (jnp.sqrt(jnp.sum(u * u)) + 1e-12)
    sigma = jnp.sum(jnp.dot(u, W, preferred_element_type=jnp.float32) * v)
    w_sn = W / sigma
    deg = (degp_ref[0, 0:N_NODES, 0:1]
           + degp_ref[1, 0:N_NODES, 0:1] + 1.0)
    d = lax.rsqrt(deg)
    g_ref[...] = jnp.dot(x_ref[...], w_sn,
                         preferred_element_type=jnp.float32) * d


def _combine_body(s_ref, g_ref, degp_ref, b_ref, out_ref):
    deg = degp_ref[0, :, 0:1] + degp_ref[1, :, 0:1] + 1.0
    d = lax.rsqrt(deg)
    out_ref[...] = d * (s_ref[0] + s_ref[1] + g_ref[...]) + b_ref[...]


_B4 = 1000


def kernel(x, edge_index, W, b, u0):
    src = edge_index[0].astype(jnp.int32).reshape(NW, EPT)
    dst = edge_index[1].astype(jnp.int32).reshape(NW, EPT)
    pad_i = jnp.arange(PADE, dtype=jnp.int32)
    pad_src = jnp.broadcast_to((pad_i * 89) % N_NODES, (NW, PADE))
    pad_dst = jnp.broadcast_to(N_NODES + pad_i, (NW, PADE))
    src_p = jnp.concatenate([src, pad_src], axis=1)
    dst_p = jnp.concatenate([dst, pad_dst], axis=1)
    packed = (dst_p * 16384 + src_p).reshape(NW, NCHUNK, CH)

    u0_2d = u0.reshape(1, D_FEAT).astype(jnp.float32)
    b_2d = b.reshape(1, D_HID).astype(jnp.float32)

    degp = _deg_kernel(packed)

    g = pl.pallas_call(
        _mm_body,
        out_shape=jax.ShapeDtypeStruct((N_NODES, D_HID), jnp.float32),
    )(x, W, u0_2d, degp)

    s = _edge_kernel(g, packed)

    out = pl.pallas_call(
        _combine_body,
        grid=(N_NODES // _B4,),
        in_specs=[
            pl.BlockSpec((NC, _B4, D_HID), lambda i: (0, i, 0)),
            pl.BlockSpec((_B4, D_HID), lambda i: (i, 0)),
            pl.BlockSpec((NC, _B4, DEGW), lambda i: (0, i, 0)),
            pl.BlockSpec((1, D_HID), lambda i: (0, 0)),
        ],
        out_specs=pl.BlockSpec((_B4, D_HID), lambda i: (i, 0)),
        out_shape=jax.ShapeDtypeStruct((N_NODES, D_HID), jnp.float32),
    )(s, g, degp, b_2d)
    return out


# speedup vs baseline: 35.9714x; 1.0710x over previous
import functools

import jax
import jax.numpy as jnp
from jax import lax
from jax.experimental import pallas as pl
from jax.experimental.pallas import tpu as pltpu
from jax.experimental.pallas import tpu_sc as plsc

N_NODES = 10000
N_EDGES = 320000
D_FEAT = 128
D_HID = 128

NC = 2
NS = 16
NW = NC * NS
EPT = N_EDGES // NW
CH = 128
NCHUNK = 79
EPT_P = NCHUNK * CH
PADE = EPT_P - EPT
N_PAD = 10112
ROWS_PT = N_PAD // NS
DEGW = 16

_mesh = plsc.VectorSubcoreMesh(core_axis_name="c", subcore_axis_name="s")


def _unpack(packed_v, j, srcc, dstc, b):
    for k in range(CH // 16):
        w = packed_v[j, pl.ds(16 * k, 16)]
        dstc[b, pl.ds(16 * k, 16)] = jnp.right_shift(w, 14)
        if srcc is not None:
            srcc[b, pl.ds(16 * k, 16)] = jnp.bitwise_and(w, 16383)


@functools.partial(
    pl.kernel,
    out_type=jax.ShapeDtypeStruct((NC, N_PAD, DEGW), jnp.float32),
    mesh=_mesh,
    scratch_types=[
        pltpu.VMEM((NCHUNK, CH), jnp.int32),
        pltpu.VMEM((CH, DEGW), jnp.float32),
        pltpu.VMEM((32, DEGW), jnp.float32),
        pltpu.VMEM((2, CH), jnp.int32),
        pltpu.VMEM_SHARED((N_PAD, DEGW), jnp.float32),
        [pltpu.SemaphoreType.DMA] * 2,
    ],
)
def _deg_kernel(pk_hbm, out_hbm, packed_v, ones_v, zeros_v, dstc, acc, ssems):
    cid = lax.axis_index("c")
    sid = lax.axis_index("s")
    wid = cid * NS + sid

    one16 = jnp.ones((16,), jnp.float32)
    zero16 = jnp.zeros((16,), jnp.float32)
    for i in range(CH):
        ones_v[i, :] = one16
    for i in range(32):
        zeros_v[i, :] = zero16
    base = sid * ROWS_PT

    def zbody(j, carry):
        pltpu.sync_copy(zeros_v, acc.at[pl.ds(base + j * 32, 32)])
        return carry

    lax.fori_loop(0, 19, zbody, 0)
    pltpu.sync_copy(zeros_v.at[pl.ds(0, 24)],
                    acc.at[pl.ds(base + 608, 24)])
    pltpu.sync_copy(pk_hbm.at[wid], packed_v)
    plsc.subcore_barrier()

    def body(j, carry):
        _unpack(packed_v, j, None, dstc, 0)
        pltpu.sync_copy(ones_v, acc.at[dstc.at[0]], add=True)
        return carry

    lax.fori_loop(0, NCHUNK, body, 0)
    plsc.subcore_barrier()
    pltpu.sync_copy(acc.at[pl.ds(base, ROWS_PT)],
                    out_hbm.at[cid, pl.ds(base, ROWS_PT)])


@functools.partial(
    pl.kernel,
    out_type=jax.ShapeDtypeStruct((NC, N_PAD, D_HID), jnp.float32),
    mesh=_mesh,
    scratch_types=[
        pltpu.VMEM((NCHUNK, CH), jnp.int32),
        pltpu.VMEM((2, CH), jnp.int32),
        pltpu.VMEM((2, CH), jnp.int32),
        pltpu.VMEM((2, CH, D_HID), jnp.float32),
        pltpu.VMEM((32, D_HID), jnp.float32),
        pltpu.VMEM_SHARED((N_PAD, D_HID), jnp.float32),
        [pltpu.SemaphoreType.DMA] * 2,
    ],
)
def _edge_kernel(g_hbm, pk_hbm, out_hbm,
                 packed_v, srcc, dstc, rows_v, zeros_v, acc, gsems):
    cid = lax.axis_index("c")
    sid = lax.axis_index("s")
    wid = cid * NS + sid

    zero16 = jnp.zeros((16,), jnp.float32)
    for i in range(32):
        for k in range(D_HID // 16):
            zeros_v[i, pl.ds(16 * k, 16)] = zero16
    base = sid * ROWS_PT

    def zbody(j, carry):
        pltpu.sync_copy(zeros_v, acc.at[pl.ds(base + j * 32, 32)])
        return carry

    lax.fori_loop(0, 19, zbody, 0)
    pltpu.sync_copy(zeros_v.at[pl.ds(0, 24)],
                    acc.at[pl.ds(base + 608, 24)])
    pltpu.sync_copy(pk_hbm.at[wid], packed_v)
    plsc.subcore_barrier()

    def gather(j, b):
        _unpack(packed_v, j, srcc, dstc, b)
        pltpu.async_copy(g_hbm.at[srcc.at[b]], rows_v.at[b], gsems[b])

    def gwait(b):
        pltpu.make_async_copy(g_hbm.at[pl.ds(0, CH)], rows_v.at[b],
                              gsems[b]).wait()

    def scatter(b):
        pltpu.sync_copy(rows_v.at[b], acc.at[dstc.at[b]], add=True)

    gather(0, 0)

    def body(t, carry):
        j = 2 * t
        gwait(0)
        gather(j + 1, 1)
        scatter(0)
        gwait(1)
        gather(j + 2, 0)
        scatter(1)
        return carry

    lax.fori_loop(0, (NCHUNK - 1) // 2, body, 0)
    gwait(0)
    scatter(0)
    plsc.subcore_barrier()
    pltpu.sync_copy(acc.at[pl.ds(base, ROWS_PT)],
                    out_hbm.at[cid, pl.ds(base, ROWS_PT)])


def _mm_body(x_ref, w_ref, u_ref, h_ref):
    W = w_ref[...]
    u = u_ref[...]
    v = None
    for _ in range(3):
        v = jnp.dot(u, W, preferred_element_type=jnp.float32)
        v = v / (jnp.sqrt(jnp.sum(v * v)) + 1e-12)
        u = lax.dot_general(v, W, (((1,), (1,)), ((), ())),
                            preferred_element_type=jnp.float32)
        u = u / (jnp.sqrt(jnp.sum(u * u)) + 1e-12)
    sigma = jnp.sum(jnp.dot(u, W, preferred_element_type=jnp.float32) * v)
    w_sn = W / sigma
    h_ref[...] = jnp.dot(x_ref[...], w_sn,
                         preferred_element_type=jnp.float32)


def _scale_body(h_ref, degp_ref, g_ref):
    deg = degp_ref[0, :, 0:1] + degp_ref[1, :, 0:1] + 1.0
    d = lax.rsqrt(deg)
    g_ref[...] = h_ref[...] * d


def _combine_body(s_ref, g_ref, degp_ref, b_ref, out_ref):
    deg = degp_ref[0, :, 0:1] + degp_ref[1, :, 0:1] + 1.0
    d = lax.rsqrt(deg)
    out_ref[...] = d * (s_ref[0] + s_ref[1] + g_ref[...]) + b_ref[...]


_B4 = 1000


def kernel(x, edge_index, W, b, u0):
    src = edge_index[0].astype(jnp.int32).reshape(NW, EPT)
    dst = edge_index[1].astype(jnp.int32).reshape(NW, EPT)
    pad_i = jnp.arange(PADE, dtype=jnp.int32)
    pad_src = jnp.broadcast_to((pad_i * 89) % N_NODES, (NW, PADE))
    pad_dst = jnp.broadcast_to(N_NODES + pad_i, (NW, PADE))
    src_p = jnp.concatenate([src, pad_src], axis=1)
    dst_p = jnp.concatenate([dst, pad_dst], axis=1)
    packed = (dst_p * 16384 + src_p).reshape(NW, NCHUNK, CH)

    u0_2d = u0.reshape(1, D_FEAT).astype(jnp.float32)
    b_2d = b.reshape(1, D_HID).astype(jnp.float32)

    degp = _deg_kernel(packed)

    h = pl.pallas_call(
        _mm_body,
        out_shape=jax.ShapeDtypeStruct((N_NODES, D_HID), jnp.float32),
    )(x, W, u0_2d)

    g = pl.pallas_call(
        _scale_body,
        grid=(N_NODES // _B4,),
        in_specs=[
            pl.BlockSpec((_B4, D_HID), lambda i: (i, 0)),
            pl.BlockSpec((NC, _B4, DEGW), lambda i: (0, i, 0)),
        ],
        out_specs=pl.BlockSpec((_B4, D_HID), lambda i: (i, 0)),
        out_shape=jax.ShapeDtypeStruct((N_NODES, D_HID), jnp.float32),
    )(h, degp)

    s = _edge_kernel(g, packed)

    out = pl.pallas_call(
        _combine_body,
        grid=(N_NODES // _B4,),
        in_specs=[
            pl.BlockSpec((NC, _B4, D_HID), lambda i: (0, i, 0)),
            pl.BlockSpec((_B4, D_HID), lambda i: (i, 0)),
            pl.BlockSpec((NC, _B4, DEGW), lambda i: (0, i, 0)),
            pl.BlockSpec((1, D_HID), lambda i: (0, 0)),
        ],
        out_specs=pl.BlockSpec((_B4, D_HID), lambda i: (i, 0)),
        out_shape=jax.ShapeDtypeStruct((N_NODES, D_HID), jnp.float32),
    )(s, g, degp, b_2d)
    return out


# speedup vs baseline: 36.0816x; 1.0031x over previous
import functools

import jax
import jax.numpy as jnp
import numpy as np
from jax import lax
from jax.experimental import pallas as pl
from jax.experimental.pallas import tpu as pltpu
from jax.experimental.pallas import tpu_sc as plsc

N_NODES = 10000
N_EDGES = 320000
D_FEAT = 128
D_HID = 128

NC = 2
NS = 16
NW = NC * NS
EPT = N_EDGES // NW
CH = 128
NCHUNK = 79
EPT_P = NCHUNK * CH
PADE = EPT_P - EPT
N_PAD = 10112
ROWS_PT = N_PAD // NS
DEGW = 16

_mesh = plsc.VectorSubcoreMesh(core_axis_name="c", subcore_axis_name="s")


def _unpack(packed_v, j, srcc, dstc, b):
    for k in range(CH // 16):
        w = packed_v[j, pl.ds(16 * k, 16)]
        dstc[b, pl.ds(16 * k, 16)] = jnp.right_shift(w, 14)
        if srcc is not None:
            srcc[b, pl.ds(16 * k, 16)] = jnp.bitwise_and(w, 16383)


@functools.partial(
    pl.kernel,
    out_type=jax.ShapeDtypeStruct((NC, N_PAD, DEGW), jnp.float32),
    mesh=_mesh,
    scratch_types=[
        pltpu.VMEM((NCHUNK, CH), jnp.int32),
        pltpu.VMEM((CH, DEGW), jnp.float32),
        pltpu.VMEM((32, DEGW), jnp.float32),
        pltpu.VMEM((2, CH), jnp.int32),
        pltpu.VMEM_SHARED((N_PAD, DEGW), jnp.float32),
        [pltpu.SemaphoreType.DMA] * 2,
    ],
)
def _deg_kernel(pk_hbm, out_hbm, packed_v, ones_v, zeros_v, dstc, acc, ssems):
    cid = lax.axis_index("c")
    sid = lax.axis_index("s")
    wid = cid * NS + sid

    one16 = jnp.ones((16,), jnp.float32)
    zero16 = jnp.zeros((16,), jnp.float32)
    for i in range(CH):
        ones_v[i, :] = one16
    for i in range(32):
        zeros_v[i, :] = zero16
    base = sid * ROWS_PT

    def zbody(j, carry):
        pltpu.sync_copy(zeros_v, acc.at[pl.ds(base + j * 32, 32)])
        return carry

    lax.fori_loop(0, 19, zbody, 0)
    pltpu.sync_copy(zeros_v.at[pl.ds(0, 24)],
                    acc.at[pl.ds(base + 608, 24)])
    pltpu.sync_copy(pk_hbm.at[wid], packed_v)
    plsc.subcore_barrier()

    def body(j, carry):
        _unpack(packed_v, j, None, dstc, 0)
        pltpu.sync_copy(ones_v, acc.at[dstc.at[0]], add=True)
        return carry

    lax.fori_loop(0, NCHUNK, body, 0)
    plsc.subcore_barrier()
    pltpu.sync_copy(acc.at[pl.ds(base, ROWS_PT)],
                    out_hbm.at[cid, pl.ds(base, ROWS_PT)])


@functools.partial(
    pl.kernel,
    out_type=jax.ShapeDtypeStruct((NC, N_PAD, D_HID), jnp.float32),
    mesh=_mesh,
    scratch_types=[
        pltpu.VMEM((NCHUNK, CH), jnp.int32),
        pltpu.VMEM((2, CH), jnp.int32),
        pltpu.VMEM((2, CH), jnp.int32),
        pltpu.VMEM((2, CH, D_HID), jnp.float32),
        pltpu.VMEM((32, D_HID), jnp.float32),
        pltpu.VMEM_SHARED((N_PAD, D_HID), jnp.float32),
        [pltpu.SemaphoreType.DMA] * 2,
    ],
)
def _edge_kernel(g_hbm, pk_hbm, out_hbm,
                 packed_v, srcc, dstc, rows_v, zeros_v, acc, gsems):
    cid = lax.axis_index("c")
    sid = lax.axis_index("s")
    wid = cid * NS + sid

    zero16 = jnp.zeros((16,), jnp.float32)
    for i in range(32):
        for k in range(D_HID // 16):
            zeros_v[i, pl.ds(16 * k, 16)] = zero16
    base = sid * ROWS_PT

    def zbody(j, carry):
        pltpu.sync_copy(zeros_v, acc.at[pl.ds(base + j * 32, 32)])
        return carry

    lax.fori_loop(0, 19, zbody, 0)
    pltpu.sync_copy(zeros_v.at[pl.ds(0, 24)],
                    acc.at[pl.ds(base + 608, 24)])
    pltpu.sync_copy(pk_hbm.at[wid], packed_v)
    plsc.subcore_barrier()

    def gather(j, b):
        _unpack(packed_v, j, srcc, dstc, b)
        pltpu.async_copy(g_hbm.at[srcc.at[b]], rows_v.at[b], gsems[b])

    def gwait(b):
        pltpu.make_async_copy(g_hbm.at[pl.ds(0, CH)], rows_v.at[b],
                              gsems[b]).wait()

    def scatter(b):
        pltpu.sync_copy(rows_v.at[b], acc.at[dstc.at[b]], add=True)

    gather(0, 0)

    def body(t, carry):
        j = 2 * t
        gwait(0)
        gather(j + 1, 1)
        scatter(0)
        gwait(1)
        gather(j + 2, 0)
        scatter(1)
        return carry

    lax.fori_loop(0, (NCHUNK - 1) // 2, body, 0)
    gwait(0)
    scatter(0)
    plsc.subcore_barrier()
    pltpu.sync_copy(acc.at[pl.ds(base, ROWS_PT)],
                    out_hbm.at[cid, pl.ds(base, ROWS_PT)])


def _mm_body(x_ref, w_ref, u_ref, h_ref):
    W = w_ref[...]
    u = u_ref[...]
    v = None
    for _ in range(3):
        v = jnp.dot(u, W, preferred_element_type=jnp.float32)
        v = v / (jnp.sqrt(jnp.sum(v * v)) + 1e-12)
        u = lax.dot_general(v, W, (((1,), (1,)), ((), ())),
                            preferred_element_type=jnp.float32)
        u = u / (jnp.sqrt(jnp.sum(u * u)) + 1e-12)
    sigma = jnp.sum(jnp.dot(u, W, preferred_element_type=jnp.float32) * v)
    w_sn = W / sigma
    h_ref[...] = jnp.dot(x_ref[...], w_sn,
                         preferred_element_type=jnp.float32)


def _scale_body(h_ref, degp_ref, g_ref):
    deg = degp_ref[0, :, 0:1] + degp_ref[1, :, 0:1] + 1.0
    d = lax.rsqrt(deg)
    g_ref[...] = h_ref[...] * d


def _combine_body(s_ref, g_ref, degp_ref, b_ref, out_ref):
    deg = degp_ref[0, :, 0:1] + degp_ref[1, :, 0:1] + 1.0
    d = lax.rsqrt(deg)
    out_ref[...] = d * (s_ref[0] + s_ref[1] + g_ref[...]) + b_ref[...]


_B4 = 1000

_PAD_I = np.arange(PADE, dtype=np.int64)
_PAD_WORDS = np.broadcast_to(
    ((N_NODES + _PAD_I) * 16384 + (_PAD_I * 89) % N_NODES).astype(np.int32),
    (NW, PADE))


def kernel(x, edge_index, W, b, u0):
    src = edge_index[0].astype(jnp.int32).reshape(NW, EPT)
    dst = edge_index[1].astype(jnp.int32).reshape(NW, EPT)
    packed = jnp.concatenate([dst * 16384 + src, jnp.asarray(_PAD_WORDS)],
                             axis=1).reshape(NW, NCHUNK, CH)

    u0_2d = u0.reshape(1, D_FEAT).astype(jnp.float32)
    b_2d = b.reshape(1, D_HID).astype(jnp.float32)

    degp = _deg_kernel(packed)

    h = pl.pallas_call(
        _mm_body,
        out_shape=jax.ShapeDtypeStruct((N_NODES, D_HID), jnp.float32),
    )(x, W, u0_2d)

    g = pl.pallas_call(
        _scale_body,
        grid=(N_NODES // _B4,),
        in_specs=[
            pl.BlockSpec((_B4, D_HID), lambda i: (i, 0)),
            pl.BlockSpec((NC, _B4, DEGW), lambda i: (0, i, 0)),
        ],
        out_specs=pl.BlockSpec((_B4, D_HID), lambda i: (i, 0)),
        out_shape=jax.ShapeDtypeStruct((N_NODES, D_HID), jnp.float32),
    )(h, degp)

    s = _edge_kernel(g, packed)

    out = pl.pallas_call(
        _combine_body,
        grid=(N_NODES // _B4,),
        in_specs=[
            pl.BlockSpec((NC, _B4, D_HID), lambda i: (0, i, 0)),
            pl.BlockSpec((_B4, D_HID), lambda i: (i, 0)),
            pl.BlockSpec((NC, _B4, DEGW), lambda i: (0, i, 0)),
            pl.BlockSpec((1, D_HID), lambda i: (0, 0)),
        ],
        out_specs=pl.BlockSpec((_B4, D_HID), lambda i: (i, 0)),
        out_shape=jax.ShapeDtypeStruct((N_NODES, D_HID), jnp.float32),
    )(s, g, degp, b_2d)
    return out


# speedup vs baseline: 36.8581x; 1.0215x over previous
import functools

import jax
import jax.numpy as jnp
import numpy as np
from jax import lax
from jax.experimental import pallas as pl
from jax.experimental.pallas import tpu as pltpu
from jax.experimental.pallas import tpu_sc as plsc

N_NODES = 10000
N_EDGES = 320000
D_FEAT = 128
D_HID = 128

NC = 2
NS = 16
NW = NC * NS
EPT = N_EDGES // NW
CH = 128
NCHUNK = 79
EPT_P = NCHUNK * CH
PADE = EPT_P - EPT
N_PAD = 10112
ROWS_PT = N_PAD // NS
DEGW = 16

_mesh = plsc.VectorSubcoreMesh(core_axis_name="c", subcore_axis_name="s")


def _unpack(packed_v, j, srcc, dstc, b):
    for k in range(CH // 16):
        w = packed_v[j, pl.ds(16 * k, 16)]
        dstc[b, pl.ds(16 * k, 16)] = jnp.right_shift(w, 14)
        if srcc is not None:
            srcc[b, pl.ds(16 * k, 16)] = jnp.bitwise_and(w, 16383)


@functools.partial(
    pl.kernel,
    out_type=jax.ShapeDtypeStruct((NC, N_PAD, DEGW), jnp.float32),
    mesh=_mesh,
    scratch_types=[
        pltpu.VMEM((NCHUNK, CH), jnp.int32),
        pltpu.VMEM((CH, DEGW), jnp.float32),
        pltpu.VMEM((32, DEGW), jnp.float32),
        pltpu.VMEM((2, CH), jnp.int32),
        pltpu.VMEM_SHARED((N_PAD, DEGW), jnp.float32),
        [pltpu.SemaphoreType.DMA] * 2,
    ],
)
def _deg_kernel(pk_hbm, out_hbm, packed_v, ones_v, zeros_v, dstc, acc, ssems):
    cid = lax.axis_index("c")
    sid = lax.axis_index("s")
    wid = cid * NS + sid

    one16 = jnp.ones((16,), jnp.float32)
    zero16 = jnp.zeros((16,), jnp.float32)
    for i in range(CH):
        ones_v[i, :] = one16
    for i in range(32):
        zeros_v[i, :] = zero16
    base = sid * ROWS_PT

    def zbody(j, carry):
        pltpu.sync_copy(zeros_v, acc.at[pl.ds(base + j * 32, 32)])
        return carry

    lax.fori_loop(0, 19, zbody, 0)
    pltpu.sync_copy(zeros_v.at[pl.ds(0, 24)],
                    acc.at[pl.ds(base + 608, 24)])
    pltpu.sync_copy(pk_hbm.at[wid], packed_v)
    plsc.subcore_barrier()

    def body(j, carry):
        _unpack(packed_v, j, None, dstc, 0)
        pltpu.sync_copy(ones_v, acc.at[dstc.at[0]], add=True)
        return carry

    lax.fori_loop(0, NCHUNK, body, 0)
    plsc.subcore_barrier()
    pltpu.sync_copy(acc.at[pl.ds(base, ROWS_PT)],
                    out_hbm.at[cid, pl.ds(base, ROWS_PT)])


@functools.partial(
    pl.kernel,
    out_type=jax.ShapeDtypeStruct((NC, N_PAD, D_HID), jnp.float32),
    mesh=_mesh,
    scratch_types=[
        pltpu.VMEM((NCHUNK, CH), jnp.int32),
        pltpu.VMEM((2, CH), jnp.int32),
        pltpu.VMEM((2, CH), jnp.int32),
        pltpu.VMEM((2, CH, D_HID), jnp.float32),
        pltpu.VMEM((32, D_HID), jnp.float32),
        pltpu.VMEM_SHARED((N_PAD, D_HID), jnp.float32),
        [pltpu.SemaphoreType.DMA] * 2,
    ],
)
def _edge_kernel(g_hbm, pk_hbm, out_hbm,
                 packed_v, srcc, dstc, rows_v, zeros_v, acc, gsems):
    cid = lax.axis_index("c")
    sid = lax.axis_index("s")
    wid = cid * NS + sid

    zero16 = jnp.zeros((16,), jnp.float32)
    for i in range(32):
        for k in range(D_HID // 16):
            zeros_v[i, pl.ds(16 * k, 16)] = zero16
    base = sid * ROWS_PT

    def zbody(j, carry):
        pltpu.sync_copy(zeros_v, acc.at[pl.ds(base + j * 32, 32)])
        return carry

    lax.fori_loop(0, 19, zbody, 0)
    pltpu.sync_copy(zeros_v.at[pl.ds(0, 24)],
                    acc.at[pl.ds(base + 608, 24)])
    pltpu.sync_copy(pk_hbm.at[wid], packed_v)
    plsc.subcore_barrier()

    def gather(j, b):
        _unpack(packed_v, j, srcc, dstc, b)
        pltpu.async_copy(g_hbm.at[srcc.at[b]], rows_v.at[b], gsems[b])

    def gwait(b):
        pltpu.make_async_copy(g_hbm.at[pl.ds(0, CH)], rows_v.at[b],
                              gsems[b]).wait()

    def scatter(b):
        pltpu.sync_copy(rows_v.at[b], acc.at[dstc.at[b]], add=True)

    gather(0, 0)

    def body(t, carry):
        j = 2 * t
        gwait(0)
        gather(j + 1, 1)
        scatter(0)
        gwait(1)
        gather(j + 2, 0)
        scatter(1)
        return carry

    lax.fori_loop(0, (NCHUNK - 1) // 2, body, 0)
    gwait(0)
    scatter(0)
    plsc.subcore_barrier()
    pltpu.sync_copy(acc.at[pl.ds(base, ROWS_PT)],
                    out_hbm.at[cid, pl.ds(base, ROWS_PT)])


def _pack_body(ei_ref, pk_ref):
    w = ei_ref[1, :, :] * 16384 + ei_ref[0, :, :]
    io = lax.broadcasted_iota(jnp.int32, (8, PADE), 1)
    padrows = (N_NODES + io) * 16384 + (io * 89) % N_NODES
    pk_ref[:, 0:EPT] = w
    pk_ref[:, EPT:EPT_P] = padrows


def _mm_body(x_ref, w_ref, u_ref, h_ref):
    W = w_ref[...]
    u = u_ref[...]
    v = None
    for _ in range(3):
        v = jnp.dot(u, W, preferred_element_type=jnp.float32)
        v = v / (jnp.sqrt(jnp.sum(v * v)) + 1e-12)
        u = lax.dot_general(v, W, (((1,), (1,)), ((), ())),
                            preferred_element_type=jnp.float32)
        u = u / (jnp.sqrt(jnp.sum(u * u)) + 1e-12)
    sigma = jnp.sum(jnp.dot(u, W, preferred_element_type=jnp.float32) * v)
    w_sn = W / sigma
    h_ref[...] = jnp.dot(x_ref[...], w_sn,
                         preferred_element_type=jnp.float32)


def _scale_body(h_ref, degp_ref, g_ref):
    deg = degp_ref[0, :, 0:1] + degp_ref[1, :, 0:1] + 1.0
    d = lax.rsqrt(deg)
    g_ref[...] = h_ref[...] * d


def _combine_body(s_ref, g_ref, degp_ref, b_ref, out_ref):
    deg = degp_ref[0, :, 0:1] + degp_ref[1, :, 0:1] + 1.0
    d = lax.rsqrt(deg)
    out_ref[...] = d * (s_ref[0] + s_ref[1] + g_ref[...]) + b_ref[...]


_B4 = 1000

_PAD_I = np.arange(PADE, dtype=np.int64)
_PAD_WORDS = np.broadcast_to(
    ((N_NODES + _PAD_I) * 16384 + (_PAD_I * 89) % N_NODES).astype(np.int32),
    (NW, PADE))


def kernel(x, edge_index, W, b, u0):
    ei = edge_index.astype(jnp.int32).reshape(2, NW, EPT)
    packed = pl.pallas_call(
        _pack_body,
        grid=(NW // 8,),
        in_specs=[pl.BlockSpec((2, 8, EPT), lambda i: (0, i, 0))],
        out_specs=pl.BlockSpec((8, EPT_P), lambda i: (i, 0)),
        out_shape=jax.ShapeDtypeStruct((NW, EPT_P), jnp.int32),
    )(ei).reshape(NW, NCHUNK, CH)

    u0_2d = u0.reshape(1, D_FEAT).astype(jnp.float32)
    b_2d = b.reshape(1, D_HID).astype(jnp.float32)

    degp = _deg_kernel(packed)

    h = pl.pallas_call(
        _mm_body,
        out_shape=jax.ShapeDtypeStruct((N_NODES, D_HID), jnp.float32),
    )(x, W, u0_2d)

    g = pl.pallas_call(
        _scale_body,
        grid=(N_NODES // _B4,),
        in_specs=[
            pl.BlockSpec((_B4, D_HID), lambda i: (i, 0)),
            pl.BlockSpec((NC, _B4, DEGW), lambda i: (0, i, 0)),
        ],
        out_specs=pl.BlockSpec((_B4, D_HID), lambda i: (i, 0)),
        out_shape=jax.ShapeDtypeStruct((N_NODES, D_HID), jnp.float32),
    )(h, degp)

    s = _edge_kernel(g, packed)

    out = pl.pallas_call(
        _combine_body,
        grid=(N_NODES // _B4,),
        in_specs=[
            pl.BlockSpec((NC, _B4, D_HID), lambda i: (0, i, 0)),
            pl.BlockSpec((_B4, D_HID), lambda i: (i, 0)),
            pl.BlockSpec((NC, _B4, DEGW), lambda i: (0, i, 0)),
            pl.BlockSpec((1, D_HID), lambda i: (0, 0)),
        ],
        out_specs=pl.BlockSpec((_B4, D_HID), lambda i: (i, 0)),
        out_shape=jax.ShapeDtypeStruct((N_NODES, D_HID), jnp.float32),
    )(s, g, degp, b_2d)
    return out


# speedup vs baseline: 37.0828x; 1.0061x over previous
import functools

import jax
import jax.numpy as jnp
import numpy as np
from jax import lax
from jax.experimental import pallas as pl
from jax.experimental.pallas import tpu as pltpu
from jax.experimental.pallas import tpu_sc as plsc

N_NODES = 10000
N_EDGES = 320000
D_FEAT = 128
D_HID = 128

NC = 2
NS = 16
NW = NC * NS
EPT = N_EDGES // NW
CH = 128
NCHUNK = 79
EPT_P = NCHUNK * CH
PADE = EPT_P - EPT
N_PAD = 10112
ROWS_PT = N_PAD // NS
DEGW = 16

_mesh = plsc.VectorSubcoreMesh(core_axis_name="c", subcore_axis_name="s")


def _unpack(packed_v, j, srcc, dstc, b):
    for k in range(CH // 16):
        w = packed_v[j, pl.ds(16 * k, 16)]
        dstc[b, pl.ds(16 * k, 16)] = jnp.right_shift(w, 14)
        if srcc is not None:
            srcc[b, pl.ds(16 * k, 16)] = jnp.bitwise_and(w, 16383)


@functools.partial(
    pl.kernel,
    out_type=jax.ShapeDtypeStruct((NC, N_PAD, DEGW), jnp.float32),
    mesh=_mesh,
    scratch_types=[
        pltpu.VMEM((NCHUNK, CH), jnp.int32),
        pltpu.VMEM((CH, DEGW), jnp.float32),
        pltpu.VMEM((32, DEGW), jnp.float32),
        pltpu.VMEM((2, CH), jnp.int32),
        pltpu.VMEM_SHARED((N_PAD, DEGW), jnp.float32),
        [pltpu.SemaphoreType.DMA] * 2,
    ],
)
def _deg_kernel(pk_hbm, out_hbm, packed_v, ones_v, zeros_v, dstc, acc, ssems):
    cid = lax.axis_index("c")
    sid = lax.axis_index("s")
    wid = cid * NS + sid

    one16 = jnp.ones((16,), jnp.float32)
    zero16 = jnp.zeros((16,), jnp.float32)
    for i in range(CH):
        ones_v[i, :] = one16
    for i in range(32):
        zeros_v[i, :] = zero16
    base = sid * ROWS_PT

    def zbody(j, carry):
        pltpu.sync_copy(zeros_v, acc.at[pl.ds(base + j * 32, 32)])
        return carry

    lax.fori_loop(0, 19, zbody, 0)
    pltpu.sync_copy(zeros_v.at[pl.ds(0, 24)],
                    acc.at[pl.ds(base + 608, 24)])
    pltpu.sync_copy(pk_hbm.at[wid], packed_v)
    plsc.subcore_barrier()

    def body(j, carry):
        _unpack(packed_v, j, None, dstc, 0)
        pltpu.sync_copy(ones_v, acc.at[dstc.at[0]], add=True)
        return carry

    lax.fori_loop(0, NCHUNK, body, 0)
    plsc.subcore_barrier()
    pltpu.sync_copy(acc.at[pl.ds(base, ROWS_PT)],
                    out_hbm.at[cid, pl.ds(base, ROWS_PT)])


@functools.partial(
    pl.kernel,
    out_type=jax.ShapeDtypeStruct((NC, N_PAD, D_HID), jnp.float32),
    mesh=_mesh,
    scratch_types=[
        pltpu.VMEM((NCHUNK, CH), jnp.int32),
        pltpu.VMEM((2, CH), jnp.int32),
        pltpu.VMEM((2, CH), jnp.int32),
        pltpu.VMEM((2, CH, D_HID), jnp.float32),
        pltpu.VMEM((32, D_HID), jnp.float32),
        pltpu.VMEM_SHARED((N_PAD, D_HID), jnp.float32),
        [pltpu.SemaphoreType.DMA] * 2,
    ],
)
def _edge_kernel(g_hbm, pk_hbm, out_hbm,
                 packed_v, srcc, dstc, rows_v, zeros_v, acc, gsems):
    cid = lax.axis_index("c")
    sid = lax.axis_index("s")
    wid = cid * NS + sid

    zero16 = jnp.zeros((16,), jnp.float32)
    for i in range(32):
        for k in range(D_HID // 16):
            zeros_v[i, pl.ds(16 * k, 16)] = zero16
    base = sid * ROWS_PT

    def zbody(j, carry):
        pltpu.sync_copy(zeros_v, acc.at[pl.ds(base + j * 32, 32)])
        return carry

    lax.fori_loop(0, 19, zbody, 0)
    pltpu.sync_copy(zeros_v.at[pl.ds(0, 24)],
                    acc.at[pl.ds(base + 608, 24)])
    pltpu.sync_copy(pk_hbm.at[wid], packed_v)
    plsc.subcore_barrier()

    def gather(j, b):
        _unpack(packed_v, j, srcc, dstc, b)
        pltpu.async_copy(g_hbm.at[srcc.at[b]], rows_v.at[b], gsems[b])

    def gwait(b):
        pltpu.make_async_copy(g_hbm.at[pl.ds(0, CH)], rows_v.at[b],
                              gsems[b]).wait()

    def scatter(b):
        pltpu.sync_copy(rows_v.at[b], acc.at[dstc.at[b]], add=True)

    gather(0, 0)

    def body(t, carry):
        j = 2 * t
        gwait(0)
        gather(j + 1, 1)
        scatter(0)
        gwait(1)
        gather(j + 2, 0)
        scatter(1)
        return carry

    lax.fori_loop(0, (NCHUNK - 1) // 2, body, 0)
    gwait(0)
    scatter(0)
    plsc.subcore_barrier()
    pltpu.sync_copy(acc.at[pl.ds(base, ROWS_PT)],
                    out_hbm.at[cid, pl.ds(base, ROWS_PT)])


def _pack_body(ei_ref, pk_ref):
    w = ei_ref[1, :, :] * 16384 + ei_ref[0, :, :]
    io = lax.broadcasted_iota(jnp.int32, (NW, PADE), 1)
    padrows = (N_NODES + io) * 16384 + (io * 89) % N_NODES
    pk_ref[...] = jnp.concatenate([w, padrows], axis=1)


def _mm_body(x_ref, w_ref, u_ref, h_ref):
    W = w_ref[...]
    u = u_ref[...]
    v = None
    for _ in range(3):
        v = jnp.dot(u, W, preferred_element_type=jnp.float32)
        v = v / (jnp.sqrt(jnp.sum(v * v)) + 1e-12)
        u = lax.dot_general(v, W, (((1,), (1,)), ((), ())),
                            preferred_element_type=jnp.float32)
        u = u / (jnp.sqrt(jnp.sum(u * u)) + 1e-12)
    sigma = jnp.sum(jnp.dot(u, W, preferred_element_type=jnp.float32) * v)
    w_sn = W / sigma
    h_ref[...] = jnp.dot(x_ref[...], w_sn,
                         preferred_element_type=jnp.float32)


def _scale_body(h_ref, degp_ref, g_ref):
    deg = degp_ref[0, :, 0:1] + degp_ref[1, :, 0:1] + 1.0
    d = lax.rsqrt(deg)
    g_ref[...] = h_ref[...] * d


def _combine_body(s_ref, g_ref, degp_ref, b_ref, out_ref):
    deg = degp_ref[0, :, 0:1] + degp_ref[1, :, 0:1] + 1.0
    d = lax.rsqrt(deg)
    out_ref[...] = d * (s_ref[0] + s_ref[1] + g_ref[...]) + b_ref[...]


_B4 = 1000

_PAD_I = np.arange(PADE, dtype=np.int64)
_PAD_WORDS = np.broadcast_to(
    ((N_NODES + _PAD_I) * 16384 + (_PAD_I * 89) % N_NODES).astype(np.int32),
    (NW, PADE))


def kernel(x, edge_index, W, b, u0):
    ei = edge_index.astype(jnp.int32).reshape(2, NW, EPT)
    packed = pl.pallas_call(
        _pack_body,
        out_shape=jax.ShapeDtypeStruct((NW, EPT_P), jnp.int32),
    )(ei).reshape(NW, NCHUNK, CH)

    u0_2d = u0.reshape(1, D_FEAT).astype(jnp.float32)
    b_2d = b.reshape(1, D_HID).astype(jnp.float32)

    degp = _deg_kernel(packed)

    h = pl.pallas_call(
        _mm_body,
        out_shape=jax.ShapeDtypeStruct((N_NODES, D_HID), jnp.float32),
    )(x, W, u0_2d)

    g = pl.pallas_call(
        _scale_body,
        grid=(N_NODES // _B4,),
        in_specs=[
            pl.BlockSpec((_B4, D_HID), lambda i: (i, 0)),
            pl.BlockSpec((NC, _B4, DEGW), lambda i: (0, i, 0)),
        ],
        out_specs=pl.BlockSpec((_B4, D_HID), lambda i: (i, 0)),
        out_shape=jax.ShapeDtypeStruct((N_NODES, D_HID), jnp.float32),
    )(h, degp)

    s = _edge_kernel(g, packed)

    out = pl.pallas_call(
        _combine_body,
        grid=(N_NODES // _B4,),
        in_specs=[
            pl.BlockSpec((NC, _B4, D_HID), lambda i: (0, i, 0)),
            pl.BlockSpec((_B4, D_HID), lambda i: (i, 0)),
            pl.BlockSpec((NC, _B4, DEGW), lambda i: (0, i, 0)),
            pl.BlockSpec((1, D_HID), lambda i: (0, 0)),
        ],
        out_specs=pl.BlockSpec((_B4, D_HID), lambda i: (i, 0)),
        out_shape=jax.ShapeDtypeStruct((N_NODES, D_HID), jnp.float32),
    )(s, g, degp, b_2d)
    return out


# speedup vs baseline: 37.1036x; 1.0006x over previous
import functools

import jax
import jax.numpy as jnp
from jax import lax
from jax.experimental import pallas as pl
from jax.experimental.pallas import tpu as pltpu
from jax.experimental.pallas import tpu_sc as plsc

N_NODES = 10000
N_EDGES = 320000
D_FEAT = 128
D_HID = 128

NC = 2
NS = 16
NW = NC * NS
EPT = N_EDGES // NW
CH = 128
NCHUNK = 79
EPT_P = NCHUNK * CH
PADE = EPT_P - EPT
N_PAD = 10112
ROWS_PT = N_PAD // NS
DEGW = 16

_mesh = plsc.VectorSubcoreMesh(core_axis_name="c", subcore_axis_name="s")


def _unpack(packed_v, j, srcc, dstc, b):
    for k in range(CH // 16):
        w = packed_v[j, pl.ds(16 * k, 16)]
        dstc[b, pl.ds(16 * k, 16)] = jnp.right_shift(w, 14)
        if srcc is not None:
            srcc[b, pl.ds(16 * k, 16)] = jnp.bitwise_and(w, 16383)


@functools.partial(
    pl.kernel,
    out_type=jax.ShapeDtypeStruct((NC, N_PAD, DEGW), jnp.float32),
    mesh=_mesh,
    scratch_types=[
        pltpu.VMEM((NCHUNK, CH), jnp.int32),
        pltpu.VMEM((CH, DEGW), jnp.float32),
        pltpu.VMEM((32, DEGW), jnp.float32),
        pltpu.VMEM((2, CH), jnp.int32),
        pltpu.VMEM_SHARED((N_PAD, DEGW), jnp.float32),
        [pltpu.SemaphoreType.DMA] * 2,
    ],
)
def _deg_kernel(pk_hbm, out_hbm, packed_v, ones_v, zeros_v, dstc, acc, ssems):
    cid = lax.axis_index("c")
    sid = lax.axis_index("s")
    wid = cid * NS + sid

    one16 = jnp.ones((16,), jnp.float32)
    zero16 = jnp.zeros((16,), jnp.float32)
    for i in range(CH):
        ones_v[i, :] = one16
    for i in range(32):
        zeros_v[i, :] = zero16
    base = sid * ROWS_PT

    def zbody(j, carry):
        pltpu.sync_copy(zeros_v, acc.at[pl.ds(base + j * 32, 32)])
        return carry

    lax.fori_loop(0, 19, zbody, 0)
    pltpu.sync_copy(zeros_v.at[pl.ds(0, 24)],
                    acc.at[pl.ds(base + 608, 24)])
    pltpu.sync_copy(pk_hbm.at[wid], packed_v)
    plsc.subcore_barrier()

    def body(j, carry):
        _unpack(packed_v, j, None, dstc, 0)
        pltpu.sync_copy(ones_v, acc.at[dstc.at[0]], add=True)
        return carry

    lax.fori_loop(0, NCHUNK, body, 0)
    plsc.subcore_barrier()
    pltpu.sync_copy(acc.at[pl.ds(base, ROWS_PT)],
                    out_hbm.at[cid, pl.ds(base, ROWS_PT)])


@functools.partial(
    pl.kernel,
    out_type=jax.ShapeDtypeStruct((NC, N_PAD, D_HID), jnp.float32),
    mesh=_mesh,
    scratch_types=[
        pltpu.VMEM((NCHUNK, CH), jnp.int32),
        pltpu.VMEM((2, CH), jnp.int32),
        pltpu.VMEM((2, CH), jnp.int32),
        pltpu.VMEM((2, CH, D_HID), jnp.float32),
        pltpu.VMEM((32, D_HID), jnp.float32),
        pltpu.VMEM_SHARED((N_PAD, D_HID), jnp.float32),
        [pltpu.SemaphoreType.DMA] * 2,
    ],
)
def _edge_kernel(g_hbm, pk_hbm, out_hbm,
                 packed_v, srcc, dstc, rows_v, zeros_v, acc, gsems):
    cid = lax.axis_index("c")
    sid = lax.axis_index("s")
    wid = cid * NS + sid

    zero16 = jnp.zeros((16,), jnp.float32)
    for i in range(32):
        for k in range(D_HID // 16):
            zeros_v[i, pl.ds(16 * k, 16)] = zero16
    base = sid * ROWS_PT

    def zbody(j, carry):
        pltpu.sync_copy(zeros_v, acc.at[pl.ds(base + j * 32, 32)])
        return carry

    lax.fori_loop(0, 19, zbody, 0)
    pltpu.sync_copy(zeros_v.at[pl.ds(0, 24)],
                    acc.at[pl.ds(base + 608, 24)])
    pltpu.sync_copy(pk_hbm.at[wid], packed_v)
    plsc.subcore_barrier()

    def gather(j, b):
        _unpack(packed_v, j, srcc, dstc, b)
        pltpu.async_copy(g_hbm.at[srcc.at[b]], rows_v.at[b], gsems[b])

    def gwait(b):
        pltpu.make_async_copy(g_hbm.at[pl.ds(0, CH)], rows_v.at[b],
                              gsems[b]).wait()

    def scatter(b):
        pltpu.sync_copy(rows_v.at[b], acc.at[dstc.at[b]], add=True)

    gather(0, 0)

    def body(t, carry):
        j = 2 * t
        gwait(0)
        gather(j + 1, 1)
        scatter(0)
        gwait(1)
        gather(j + 2, 0)
        scatter(1)
        return carry

    lax.fori_loop(0, (NCHUNK - 1) // 2, body, 0)
    gwait(0)
    scatter(0)
    plsc.subcore_barrier()
    pltpu.sync_copy(acc.at[pl.ds(base, ROWS_PT)],
                    out_hbm.at[cid, pl.ds(base, ROWS_PT)])


def _pack_body(ei_ref, pk_ref):
    w = ei_ref[1, :, :] * 16384 + ei_ref[0, :, :]
    io = lax.broadcasted_iota(jnp.int32, (NW, PADE), 1)
    padrows = (N_NODES + io) * 16384 + (io * 89) % N_NODES
    pk_ref[...] = jnp.concatenate([w, padrows], axis=1)


def _mm_body(x_ref, w_ref, u_ref, h_ref):
    W = w_ref[...]
    u = u_ref[...]
    v = None
    for _ in range(3):
        v = jnp.dot(u, W, preferred_element_type=jnp.float32)
        v = v / (jnp.sqrt(jnp.sum(v * v)) + 1e-12)
        u = lax.dot_general(v, W, (((1,), (1,)), ((), ())),
                            preferred_element_type=jnp.float32)
        u = u / (jnp.sqrt(jnp.sum(u * u)) + 1e-12)
    sigma = jnp.sum(jnp.dot(u, W, preferred_element_type=jnp.float32) * v)
    w_sn = W / sigma
    h_ref[...] = jnp.dot(x_ref[...], w_sn,
                         preferred_element_type=jnp.float32)


def _scale_body(h_ref, degp_ref, g_ref):
    deg = degp_ref[0, :, 0:1] + degp_ref[1, :, 0:1] + 1.0
    d = lax.rsqrt(deg)
    g_ref[...] = h_ref[...] * d


def _combine_body(s_ref, g_ref, degp_ref, b_ref, out_ref):
    deg = degp_ref[0, :, 0:1] + degp_ref[1, :, 0:1] + 1.0
    d = lax.rsqrt(deg)
    out_ref[...] = d * (s_ref[0] + s_ref[1] + g_ref[...]) + b_ref[...]


_B4 = 1000


def kernel(x, edge_index, W, b, u0):
    ei = edge_index.astype(jnp.int32).reshape(2, NW, EPT)
    packed = pl.pallas_call(
        _pack_body,
        out_shape=jax.ShapeDtypeStruct((NW, EPT_P), jnp.int32),
    )(ei).reshape(NW, NCHUNK, CH)

    u0_2d = u0.reshape(1, D_FEAT).astype(jnp.float32)
    b_2d = b.reshape(1, D_HID).astype(jnp.float32)

    degp = _deg_kernel(packed)

    h = pl.pallas_call(
        _mm_body,
        out_shape=jax.ShapeDtypeStruct((N_NODES, D_HID), jnp.float32),
    )(x, W, u0_2d)

    g = pl.pallas_call(
        _scale_body,
        grid=(N_NODES // _B4,),
        in_specs=[
            pl.BlockSpec((_B4, D_HID), lambda i: (i, 0)),
            pl.BlockSpec((NC, _B4, DEGW), lambda i: (0, i, 0)),
        ],
        out_specs=pl.BlockSpec((_B4, D_HID), lambda i: (i, 0)),
        out_shape=jax.ShapeDtypeStruct((N_NODES, D_HID), jnp.float32),
    )(h, degp)

    s = _edge_kernel(g, packed)

    out = pl.pallas_call(
        _combine_body,
        grid=(N_NODES // _B4,),
        in_specs=[
            pl.BlockSpec((NC, _B4, D_HID), lambda i: (0, i, 0)),
            pl.BlockSpec((_B4, D_HID), lambda i: (i, 0)),
            pl.BlockSpec((NC, _B4, DEGW), lambda i: (0, i, 0)),
            pl.BlockSpec((1, D_HID), lambda i: (0, 0)),
        ],
        out_specs=pl.BlockSpec((_B4, D_HID), lambda i: (i, 0)),
        out_shape=jax.ShapeDtypeStruct((N_NODES, D_HID), jnp.float32),
    )(s, g, degp, b_2d)
    return out
